# baseline (speedup 1.0000x reference)
import functools

import jax
import jax.numpy as jnp
from jax import lax
from jax.experimental import pallas as pl
from jax.experimental.pallas import tpu as pltpu
from jax.experimental.pallas import tpu_sc as plsc

VOCAB = 1025
DIM = 64
NPAIR = DIM // 2
BATCH = 16384
WINDOW = 200
WPAD = 208
LANES = 16
NCORE = 2
NSUB = 16
NWORK = NCORE * NSUB
NB_PER_W = BATCH // NWORK
CB = 32
NCHUNK = NB_PER_W // CB
HIMASK = -65536


def _sc_pool(tp_flat, y):
    mesh = plsc.VectorSubcoreMesh(
        core_axis_name="c", subcore_axis_name="s",
        num_cores=NCORE, num_subcores=NSUB)

    @functools.partial(
        pl.kernel,
        out_type=jax.ShapeDtypeStruct((BATCH, DIM), jnp.float32),
        mesh=mesh,
        compiler_params=pltpu.CompilerParams(
            use_tc_tiling_on_sc=False, needs_layout_passes=False),
        scratch_types=[
            pltpu.VMEM((VOCAB * NPAIR,), jnp.int32),
            pltpu.VMEM((2, CB, WPAD), jnp.int32),
            pltpu.VMEM((2, CB, DIM), jnp.float32),
            pltpu.SemaphoreType.DMA,
            pltpu.SemaphoreType.DMA,
            pltpu.SemaphoreType.DMA,
            pltpu.SemaphoreType.DMA,
        ],
    )
    def k(tp_hbm, y_hbm, out_hbm, tpv, yv2, ov2, ys0, ys1, os0, os1):
        wid = lax.axis_index("s") * NCORE + lax.axis_index("c")
        row_base = wid * NB_PER_W
        pltpu.sync_copy(tp_hbm, tpv)

        ev_idx = lax.iota(jnp.int32, LANES) * 2
        ysems = (ys0, ys1)
        osems = (os0, os1)

        def start_y(ci, buf):
            pltpu.async_copy(
                y_hbm.at[pl.ds(row_base + ci * CB, CB), :],
                yv2.at[buf, :, pl.ds(0, WINDOW)], ysems[buf])

        def do_row2(bp, carry, yv, outv):
            del carry
            r0l = bp * 2
            r1l = bp * 2 + 1
            def pair(accs, idxv, u):
                ae0, ao0, ae1, ao1 = accs
                r0 = idxv[u]
                r1 = idxv[u + 1]
                b0 = pl.multiple_of(r0 * NPAIR, LANES)
                b1 = pl.multiple_of(r1 * NPAIR, LANES)
                v0a = tpv[pl.ds(b0, LANES)]
                v1a = tpv[pl.ds(pl.multiple_of(b0 + LANES, LANES), LANES)]
                v0b = tpv[pl.ds(b1, LANES)]
                v1b = tpv[pl.ds(pl.multiple_of(b1 + LANES, LANES), LANES)]
                s0 = plsc.bitcast(
                    plsc.bitcast(v0a, jnp.bfloat16)
                    + plsc.bitcast(v0b, jnp.bfloat16), jnp.int32)
                s1 = plsc.bitcast(
                    plsc.bitcast(v1a, jnp.bfloat16)
                    + plsc.bitcast(v1b, jnp.bfloat16), jnp.int32)
                ae0 = ae0 + lax.bitcast_convert_type(s0 << 16, jnp.float32)
                ao0 = ao0 + lax.bitcast_convert_type(s0 & HIMASK,
                                                     jnp.float32)
                ae1 = ae1 + lax.bitcast_convert_type(s1 << 16, jnp.float32)
                ao1 = ao1 + lax.bitcast_convert_type(s1 & HIMASK,
                                                     jnp.float32)
                return ae0, ao0, ae1, ao1

            def do_w(i, accs8):
                a, b = accs8[:4], accs8[4:]
                off = pl.multiple_of(i * LANES, LANES)
                idxv0 = yv[r0l, pl.ds(off, LANES)]
                idxv1 = yv[r1l, pl.ds(off, LANES)]
                for u in range(0, LANES, 2):
                    a = pair(a, idxv0, u)
                    b = pair(b, idxv1, u)
                return (*a, *b)

            z = jnp.zeros((LANES,), jnp.float32)
            accs8 = lax.fori_loop(
                0, WINDOW // LANES, do_w, (z,) * 8)
            a, b = accs8[:4], accs8[4:]
            idxv0 = yv[r0l, pl.ds(12 * LANES, LANES)]
            idxv1 = yv[r1l, pl.ds(12 * LANES, LANES)]
            for u in range(0, WINDOW - 12 * LANES, 2):
                a = pair(a, idxv0, u)
                b = pair(b, idxv1, u)

            sc = 1.0 / WINDOW
            for rl, (ae0, ao0, ae1, ao1) in ((r0l, a), (r1l, b)):
                blv = jnp.broadcast_to(rl, (LANES,))
                plsc.store_scatter(outv, [blv, ev_idx], ae0 * sc)
                plsc.store_scatter(outv, [blv, ev_idx + 1], ao0 * sc)
                plsc.store_scatter(outv, [blv, ev_idx + NPAIR], ae1 * sc)
                plsc.store_scatter(outv, [blv, ev_idx + (NPAIR + 1)],
                                   ao1 * sc)
            return 0

        start_y(0, 0)
        start_y(1, 1)

        def do_pair(cp, _):
            for buf in (0, 1):
                ci = cp * 2 + buf
                yv = yv2.at[buf]
                outv = ov2.at[buf]
                pltpu.make_async_copy(
                    y_hbm.at[pl.ds(row_base + ci * CB, CB), :],
                    yv2.at[buf, :, pl.ds(0, WINDOW)], ysems[buf]).wait()

                @pl.when(ci >= 2)
                def _():
                    pltpu.make_async_copy(
                        ov2.at[buf],
                        out_hbm.at[pl.ds(row_base + (ci - 2) * CB, CB), :],
                        osems[buf]).wait()

                lax.fori_loop(0, CB // 2, functools.partial(
                    do_row2, yv=yv, outv=outv), 0)
                pltpu.async_copy(
                    ov2.at[buf],
                    out_hbm.at[pl.ds(row_base + ci * CB, CB), :],
                    osems[buf])

                @pl.when(ci + 2 < NCHUNK)
                def _():
                    start_y(ci + 2, buf)
            return 0

        lax.fori_loop(0, NCHUNK // 2, do_pair, 0)
        for buf in (0, 1):
            pltpu.make_async_copy(
                ov2.at[buf],
                out_hbm.at[pl.ds(row_base + (NCHUNK - 2 + buf) * CB, CB), :],
                osems[buf]).wait()

    return k(tp_flat, y)


def kernel(y, table):
    t0 = table.at[0].set(0.0)
    tb = t0.astype(jnp.bfloat16).reshape(VOCAB, NPAIR, 2)
    tp = lax.bitcast_convert_type(tb, jnp.int32).reshape(VOCAB * NPAIR)
    return _sc_pool(tp, y.astype(jnp.int32))


# speedup vs baseline: 1.1808x; 1.1808x over previous
import functools

import jax
import jax.numpy as jnp
from jax import lax
from jax.experimental import pallas as pl
from jax.experimental.pallas import tpu as pltpu
from jax.experimental.pallas import tpu_sc as plsc

VOCAB = 1025
DIM = 64
NPAIR = DIM // 2
BATCH = 16384
WINDOW = 200
WPAD = 208
LANES = 16
NCORE = 2
NSUB = 16
NWORK = NCORE * NSUB
NB_PER_W = BATCH // NWORK
CB = 32
NCHUNK = NB_PER_W // CB
HIMASK = -65536


def _sc_pool(tp_flat, y):
    mesh = plsc.VectorSubcoreMesh(
        core_axis_name="c", subcore_axis_name="s",
        num_cores=NCORE, num_subcores=NSUB)

    @functools.partial(
        pl.kernel,
        out_type=jax.ShapeDtypeStruct((BATCH, DIM), jnp.float32),
        mesh=mesh,
        compiler_params=pltpu.CompilerParams(
            use_tc_tiling_on_sc=False, needs_layout_passes=False),
        scratch_types=[
            pltpu.VMEM((VOCAB * NPAIR,), jnp.int32),
            pltpu.VMEM((2, CB, WPAD), jnp.int32),
            pltpu.VMEM((2, CB, DIM), jnp.float32),
            pltpu.SemaphoreType.DMA,
            pltpu.SemaphoreType.DMA,
            pltpu.SemaphoreType.DMA,
            pltpu.SemaphoreType.DMA,
        ],
    )
    def k(tp_hbm, y_hbm, out_hbm, tpv, yv2, ov2, ys0, ys1, os0, os1):
        wid = lax.axis_index("s") * NCORE + lax.axis_index("c")
        row_base = wid * NB_PER_W
        pltpu.sync_copy(tp_hbm, tpv)

        ev_idx = lax.iota(jnp.int32, LANES) * 2
        ysems = (ys0, ys1)
        osems = (os0, os1)

        def start_y(ci, buf):
            pltpu.async_copy(
                y_hbm.at[pl.ds(row_base + ci * CB, CB), :],
                yv2.at[buf, :, pl.ds(0, WINDOW)], ysems[buf])

        def do_row(bl, carry, yv, outv):
            del carry
            def pair(accs, idxv, u):
                ae0, ao0, ae1, ao1 = accs
                r0 = idxv[u]
                r1 = idxv[u + 1]
                b0 = pl.multiple_of(r0 * NPAIR, LANES)
                b1 = pl.multiple_of(r1 * NPAIR, LANES)
                v0a = tpv[pl.ds(b0, LANES)]
                v1a = tpv[pl.ds(pl.multiple_of(b0 + LANES, LANES), LANES)]
                v0b = tpv[pl.ds(b1, LANES)]
                v1b = tpv[pl.ds(pl.multiple_of(b1 + LANES, LANES), LANES)]
                s0 = plsc.bitcast(
                    plsc.bitcast(v0a, jnp.bfloat16)
                    + plsc.bitcast(v0b, jnp.bfloat16), jnp.int32)
                s1 = plsc.bitcast(
                    plsc.bitcast(v1a, jnp.bfloat16)
                    + plsc.bitcast(v1b, jnp.bfloat16), jnp.int32)
                ae0 = ae0 + lax.bitcast_convert_type(s0 << 16, jnp.float32)
                ao0 = ao0 + lax.bitcast_convert_type(s0 & HIMASK,
                                                     jnp.float32)
                ae1 = ae1 + lax.bitcast_convert_type(s1 << 16, jnp.float32)
                ao1 = ao1 + lax.bitcast_convert_type(s1 & HIMASK,
                                                     jnp.float32)
                return ae0, ao0, ae1, ao1

            def do_w(i, accs8):
                a, b = accs8[:4], accs8[4:]
                idxv = yv[bl, pl.ds(pl.multiple_of(i * LANES, LANES),
                                    LANES)]
                for u in range(0, LANES, 4):
                    a = pair(a, idxv, u)
                    b = pair(b, idxv, u + 2)
                return (*a, *b)

            z = jnp.zeros((LANES,), jnp.float32)
            accs8 = lax.fori_loop(
                0, WINDOW // LANES, do_w, (z,) * 8)
            a, b = accs8[:4], accs8[4:]
            idxv = yv[bl, pl.ds(12 * LANES, LANES)]
            for u in range(0, WINDOW - 12 * LANES, 4):
                a = pair(a, idxv, u)
                b = pair(b, idxv, u + 2)
            ae0, ao0, ae1, ao1 = (x + y for x, y in zip(a, b))

            blv = jnp.broadcast_to(bl, (LANES,))
            sc = 1.0 / WINDOW
            plsc.store_scatter(outv, [blv, ev_idx], ae0 * sc)
            plsc.store_scatter(outv, [blv, ev_idx + 1], ao0 * sc)
            plsc.store_scatter(outv, [blv, ev_idx + NPAIR], ae1 * sc)
            plsc.store_scatter(outv, [blv, ev_idx + (NPAIR + 1)], ao1 * sc)
            return 0

        start_y(0, 0)
        start_y(1, 1)

        def do_pair(cp, _):
            for buf in (0, 1):
                ci = cp * 2 + buf
                yv = yv2.at[buf]
                outv = ov2.at[buf]
                pltpu.make_async_copy(
                    y_hbm.at[pl.ds(row_base + ci * CB, CB), :],
                    yv2.at[buf, :, pl.ds(0, WINDOW)], ysems[buf]).wait()

                @pl.when(ci >= 2)
                def _():
                    pltpu.make_async_copy(
                        ov2.at[buf],
                        out_hbm.at[pl.ds(row_base + (ci - 2) * CB, CB), :],
                        osems[buf]).wait()

                lax.fori_loop(0, CB, functools.partial(
                    do_row, yv=yv, outv=outv), 0)
                pltpu.async_copy(
                    ov2.at[buf],
                    out_hbm.at[pl.ds(row_base + ci * CB, CB), :],
                    osems[buf])

                @pl.when(ci + 2 < NCHUNK)
                def _():
                    start_y(ci + 2, buf)
            return 0

        lax.fori_loop(0, NCHUNK // 2, do_pair, 0)
        for buf in (0, 1):
            pltpu.make_async_copy(
                ov2.at[buf],
                out_hbm.at[pl.ds(row_base + (NCHUNK - 2 + buf) * CB, CB), :],
                osems[buf]).wait()

    return k(tp_flat, y)


def kernel(y, table):
    t0 = table.at[0].set(0.0)
    tb = t0.astype(jnp.bfloat16).reshape(VOCAB, NPAIR, 2)
    tp = lax.bitcast_convert_type(tb, jnp.int32).reshape(VOCAB * NPAIR)
    return _sc_pool(tp, y.astype(jnp.int32))


# speedup vs baseline: 1.2148x; 1.0288x over previous
import functools

import jax
import jax.numpy as jnp
from jax import lax
from jax.experimental import pallas as pl
from jax.experimental.pallas import tpu as pltpu
from jax.experimental.pallas import tpu_sc as plsc

VOCAB = 1025
DIM = 64
NPAIR = DIM // 2
BATCH = 16384
WINDOW = 200
WPAD = 208
LANES = 16
NCORE = 2
NSUB = 16
NWORK = NCORE * NSUB
NB_PER_W = BATCH // NWORK
CB = 32
NCHUNK = NB_PER_W // CB
HIMASK = -65536


def _sc_pool(tp_flat, y):
    mesh = plsc.VectorSubcoreMesh(
        core_axis_name="c", subcore_axis_name="s",
        num_cores=NCORE, num_subcores=NSUB)

    @functools.partial(
        pl.kernel,
        out_type=jax.ShapeDtypeStruct((BATCH, DIM), jnp.float32),
        mesh=mesh,
        compiler_params=pltpu.CompilerParams(
            use_tc_tiling_on_sc=False, needs_layout_passes=False),
        scratch_types=[
            pltpu.VMEM((VOCAB * NPAIR,), jnp.int32),
            pltpu.VMEM((2, CB, WPAD), jnp.int32),
            pltpu.VMEM((2, CB, DIM), jnp.float32),
            pltpu.SemaphoreType.DMA,
            pltpu.SemaphoreType.DMA,
            pltpu.SemaphoreType.DMA,
            pltpu.SemaphoreType.DMA,
        ],
    )
    def k(tp_hbm, y_hbm, out_hbm, tpv, yv2, ov2, ys0, ys1, os0, os1):
        wid = lax.axis_index("s") * NCORE + lax.axis_index("c")
        row_base = wid * NB_PER_W
        pltpu.sync_copy(tp_hbm, tpv)

        ev_idx = lax.iota(jnp.int32, LANES) * 2
        ysems = (ys0, ys1)
        osems = (os0, os1)

        def start_y(ci, buf):
            pltpu.async_copy(
                y_hbm.at[pl.ds(row_base + ci * CB, CB), :],
                yv2.at[buf, :, pl.ds(0, WINDOW)], ysems[buf])

        def do_row(bl, carry, yv, outv):
            del carry
            def pair(accs, idxv, u):
                ae0, ao0, ae1, ao1 = accs
                lo, hi = [], []
                for q in range(4):
                    b = pl.multiple_of(idxv[u + q] * NPAIR, LANES)
                    lo.append(plsc.bitcast(
                        tpv[pl.ds(b, LANES)], jnp.bfloat16))
                    hi.append(plsc.bitcast(
                        tpv[pl.ds(pl.multiple_of(b + LANES, LANES), LANES)],
                        jnp.bfloat16))
                s0 = plsc.bitcast((lo[0] + lo[1]) + (lo[2] + lo[3]),
                                  jnp.int32)
                s1 = plsc.bitcast((hi[0] + hi[1]) + (hi[2] + hi[3]),
                                  jnp.int32)
                ae0 = ae0 + lax.bitcast_convert_type(s0 << 16, jnp.float32)
                ao0 = ao0 + lax.bitcast_convert_type(s0 & HIMASK,
                                                     jnp.float32)
                ae1 = ae1 + lax.bitcast_convert_type(s1 << 16, jnp.float32)
                ao1 = ao1 + lax.bitcast_convert_type(s1 & HIMASK,
                                                     jnp.float32)
                return ae0, ao0, ae1, ao1

            def do_w(i, accs8):
                a, b = accs8[:4], accs8[4:]
                idxv = yv[bl, pl.ds(pl.multiple_of(i * LANES, LANES),
                                    LANES)]
                for u in range(0, LANES, 8):
                    a = pair(a, idxv, u)
                    b = pair(b, idxv, u + 4)
                return (*a, *b)

            z = jnp.zeros((LANES,), jnp.float32)
            accs8 = lax.fori_loop(
                0, WINDOW // LANES, do_w, (z,) * 8)
            a, b = accs8[:4], accs8[4:]
            idxv = yv[bl, pl.ds(12 * LANES, LANES)]
            for u in range(0, WINDOW - 12 * LANES, 8):
                a = pair(a, idxv, u)
                b = pair(b, idxv, u + 4)
            ae0, ao0, ae1, ao1 = (x + y for x, y in zip(a, b))

            blv = jnp.broadcast_to(bl, (LANES,))
            sc = 1.0 / WINDOW
            plsc.store_scatter(outv, [blv, ev_idx], ae0 * sc)
            plsc.store_scatter(outv, [blv, ev_idx + 1], ao0 * sc)
            plsc.store_scatter(outv, [blv, ev_idx + NPAIR], ae1 * sc)
            plsc.store_scatter(outv, [blv, ev_idx + (NPAIR + 1)], ao1 * sc)
            return 0

        start_y(0, 0)
        start_y(1, 1)

        def do_pair(cp, _):
            for buf in (0, 1):
                ci = cp * 2 + buf
                yv = yv2.at[buf]
                outv = ov2.at[buf]
                pltpu.make_async_copy(
                    y_hbm.at[pl.ds(row_base + ci * CB, CB), :],
                    yv2.at[buf, :, pl.ds(0, WINDOW)], ysems[buf]).wait()

                @pl.when(ci >= 2)
                def _():
                    pltpu.make_async_copy(
                        ov2.at[buf],
                        out_hbm.at[pl.ds(row_base + (ci - 2) * CB, CB), :],
                        osems[buf]).wait()

                lax.fori_loop(0, CB, functools.partial(
                    do_row, yv=yv, outv=outv), 0)
                pltpu.async_copy(
                    ov2.at[buf],
                    out_hbm.at[pl.ds(row_base + ci * CB, CB), :],
                    osems[buf])

                @pl.when(ci + 2 < NCHUNK)
                def _():
                    start_y(ci + 2, buf)
            return 0

        lax.fori_loop(0, NCHUNK // 2, do_pair, 0)
        for buf in (0, 1):
            pltpu.make_async_copy(
                ov2.at[buf],
                out_hbm.at[pl.ds(row_base + (NCHUNK - 2 + buf) * CB, CB), :],
                osems[buf]).wait()

    return k(tp_flat, y)


def kernel(y, table):
    t0 = table.at[0].set(0.0)
    tb = t0.astype(jnp.bfloat16).reshape(VOCAB, NPAIR, 2)
    tp = lax.bitcast_convert_type(tb, jnp.int32).reshape(VOCAB * NPAIR)
    return _sc_pool(tp, y.astype(jnp.int32))
